# baseline (speedup 1.0000x reference)
import functools

import jax
import jax.numpy as jnp
from jax import lax
from jax.experimental import pallas as pl
from jax.experimental.pallas import tpu as pltpu
from jax.experimental.pallas import tpu_sc as plsc

E = 8
K = 2
T = 2048
H = 2048
I = 1024
BLK = 256
NB = (T * K) // BLK + E
P = NB * BLK
NC = 2
NS = 16
NW = NC * NS
A = T * K
CH = A // (NW * 16)

_MESH = dict(core_axis_name="c", subcore_axis_name="s", num_cores=NC,
             num_subcores=NS)


def _router_body(logits_ref, sel_ref, ids_ref, gw_ref):
    lg = logits_ref[...]
    eio = lax.broadcasted_iota(jnp.int32, (T, E), 1)
    m1 = jnp.max(lg, axis=1, keepdims=True)
    i1 = jnp.min(jnp.where(lg == m1, eio, E), axis=1, keepdims=True)
    lg2 = jnp.where(eio == i1, -jnp.inf, lg)
    m2 = jnp.max(lg2, axis=1, keepdims=True)
    i2 = jnp.min(jnp.where(lg2 == m2, eio, E), axis=1, keepdims=True)
    p = jnp.exp(lg - m1)
    p = p / jnp.sum(p, axis=1, keepdims=True)
    sel1 = eio == i1
    sel2 = eio == i2
    g1 = jnp.sum(jnp.where(sel1, p, 0.0), axis=1, keepdims=True)
    g2 = jnp.sum(jnp.where(sel2, p, 0.0), axis=1, keepdims=True)
    sel_ref[...] = (sel1 | sel2).astype(jnp.float32)
    ids_ref[...] = jnp.concatenate([i1, i2], axis=1)
    gw_ref[...] = jnp.concatenate([g1, g2], axis=1)


_router = pl.pallas_call(
    _router_body,
    out_shape=(
        jax.ShapeDtypeStruct((T, E), jnp.float32),
        jax.ShapeDtypeStruct((T, K), jnp.int32),
        jax.ShapeDtypeStruct((T, K), jnp.float32),
    ),
)


TC = T // NW
DC = TC // 16


@functools.partial(
    pl.kernel,
    out_type=jax.ShapeDtypeStruct((P, H), jnp.float32),
    mesh=plsc.VectorSubcoreMesh(**_MESH),
    scratch_types=[
        pltpu.VMEM((DC, 16), jnp.int32),
        pltpu.VMEM((DC, 16), jnp.int32),
        pltpu.VMEM((2, 16, H), jnp.float32),
        pltpu.SemaphoreType.DMA,
        pltpu.SemaphoreType.DMA,
    ],
)
def _dispatch(hs_hbm, slot0_hbm, slot1_hbm, xs_hbm, slot0_v, slot1_v, buf,
              sem_g, sem_s):
    wid = lax.axis_index("s") * NC + lax.axis_index("c")
    pltpu.sync_copy(slot0_hbm.at[wid], slot0_v)
    pltpu.sync_copy(slot1_hbm.at[wid], slot1_v)
    g = [None] * DC
    s0 = [None] * DC
    s1 = [None] * DC
    base = wid * TC
    g[0] = pltpu.async_copy(hs_hbm.at[pl.ds(base, 16)], buf.at[0], sem_g)
    for j in range(DC):
        if j + 1 < DC:
            if j >= 1:
                s0[j - 1].wait()
                s1[j - 1].wait()
            g[j + 1] = pltpu.async_copy(
                hs_hbm.at[pl.ds(base + (j + 1) * 16, 16)],
                buf.at[(j + 1) % 2], sem_g)
        g[j].wait()
        s0[j] = pltpu.async_copy(buf.at[j % 2], xs_hbm.at[slot0_v.at[j]],
                                 sem_s)
        s1[j] = pltpu.async_copy(buf.at[j % 2], xs_hbm.at[slot1_v.at[j]],
                                 sem_s)
    s0[DC - 2].wait()
    s1[DC - 2].wait()
    s0[DC - 1].wait()
    s1[DC - 1].wait()


def _ffn_body(be_ref, xi_ref, na_ref, x_ref, w1_ref, w3_ref, w2_ref, y_ref):
    i = pl.program_id(0)

    @pl.when(i < na_ref[0])
    def _():
        x = x_ref[...].astype(jnp.bfloat16)
        a = jnp.dot(x, w1_ref[0], preferred_element_type=jnp.float32)
        b = jnp.dot(x, w3_ref[0], preferred_element_type=jnp.float32)
        h = (a * lax.logistic(a) * b).astype(jnp.bfloat16)
        y_ref[...] = jnp.dot(h, w2_ref[0], preferred_element_type=jnp.float32)


_ffn = pl.pallas_call(
    _ffn_body,
    grid_spec=pltpu.PrefetchScalarGridSpec(
        num_scalar_prefetch=3,
        grid=(NB,),
        in_specs=[
            pl.BlockSpec((BLK, H), lambda i, be, xi, na: (xi[i], 0)),
            pl.BlockSpec((1, H, I), lambda i, be, xi, na: (be[i], 0, 0)),
            pl.BlockSpec((1, H, I), lambda i, be, xi, na: (be[i], 0, 0)),
            pl.BlockSpec((1, I, H), lambda i, be, xi, na: (be[i], 0, 0)),
        ],
        out_specs=pl.BlockSpec((BLK, H), lambda i, be, xi, na: (i, 0)),
    ),
    out_shape=jax.ShapeDtypeStruct((P, H), jnp.float32),
)


@functools.partial(
    pl.kernel,
    out_type=jax.ShapeDtypeStruct((T, H), jnp.float32),
    mesh=plsc.VectorSubcoreMesh(**_MESH),
    scratch_types=[
        pltpu.VMEM((CH, 16), jnp.int32),
        pltpu.VMEM((CH, 16), jnp.float32),
        pltpu.VMEM((2, 16, H), jnp.float32),
        pltpu.VMEM((2, 8, H), jnp.float32),
        pltpu.SemaphoreType.DMA,
        pltpu.SemaphoreType.DMA,
    ],
)
def _combine(y_hbm, slot_hbm, gate_hbm, out_hbm, slot_v, g_v, ybuf, obuf,
             sem_y, sem_o):
    wid = lax.axis_index("s") * NC + lax.axis_index("c")
    pltpu.sync_copy(slot_hbm.at[wid], slot_v)
    pltpu.sync_copy(gate_hbm.at[wid], g_v)
    yd = [None] * CH
    od = [None] * CH
    yd[0] = pltpu.async_copy(y_hbm.at[slot_v.at[0]], ybuf.at[0], sem_y)
    for j in range(CH):
        if j + 1 < CH:
            yd[j + 1] = pltpu.async_copy(
                y_hbm.at[slot_v.at[j + 1]], ybuf.at[(j + 1) % 2], sem_y)
        yd[j].wait()
        if j >= 2:
            od[j - 2].wait()
        jb = j % 2
        gvec = g_v[j]
        gs = [gvec[i] for i in range(16)]

        def body(c, _, jb=jb, gs=gs):
            sl = pl.ds(c * 16, 16)
            for r in range(8):
                obuf[jb, r, sl] = (gs[2 * r] * ybuf[jb, 2 * r, sl]
                                   + gs[2 * r + 1] * ybuf[jb, 2 * r + 1, sl])
            return 0

        lax.fori_loop(0, H // 16, body, 0)
        od[j] = pltpu.async_copy(
            obuf.at[jb], out_hbm.at[pl.ds(wid * (T // NW) + j * 8, 8)], sem_o)
    od[CH - 2].wait()
    od[CH - 1].wait()


def kernel(hidden_states, router_logits, w1, w3, w2):
    sel, ids, gw = _router(router_logits)

    rank = jnp.take_along_axis(jnp.cumsum(sel, axis=0) - sel, ids, axis=1)
    counts = jnp.sum(sel, axis=0).astype(jnp.int32)
    nblk = (counts + BLK - 1) // BLK
    blk_end = jnp.cumsum(nblk)
    blk_start = blk_end - nblk
    num_active = blk_end[-1]
    j_eff = jnp.minimum(jnp.arange(NB, dtype=jnp.int32), num_active - 1)
    block_expert = jnp.searchsorted(blk_end, j_eff, side="right").astype(
        jnp.int32)
    slot = blk_start[ids] * BLK + rank.astype(jnp.int32)

    slot3 = slot.reshape(NW, CH, 16)
    slot0_3 = slot[:, 0].reshape(NW, DC, 16)
    slot1_3 = slot[:, 1].reshape(NW, DC, 16)
    gate3 = gw.reshape(NW, CH, 16)

    x_sorted = _dispatch(hidden_states, slot0_3, slot1_3)
    y = _ffn(block_expert, j_eff, num_active[None], x_sorted,
             w1.astype(jnp.bfloat16), w3.astype(jnp.bfloat16),
             w2.astype(jnp.bfloat16))
    return _combine(y, slot3, gate3)


# speedup vs baseline: 1.2984x; 1.2984x over previous
import functools

import jax
import jax.numpy as jnp
from jax import lax
from jax.experimental import pallas as pl
from jax.experimental.pallas import tpu as pltpu
from jax.experimental.pallas import tpu_sc as plsc

E = 8
K = 2
T = 2048
H = 2048
I = 1024
BLK = 256
NB = (T * K) // BLK + E
P = NB * BLK
NC = 2
NS = 16
NW = NC * NS
A = T * K
CH = A // (NW * 16)

_MESH = dict(core_axis_name="c", subcore_axis_name="s", num_cores=NC,
             num_subcores=NS)


def _router_body(logits_ref, sel_ref, ids_ref, gw_ref):
    lg = logits_ref[...]
    eio = lax.broadcasted_iota(jnp.int32, (T, E), 1)
    m1 = jnp.max(lg, axis=1, keepdims=True)
    i1 = jnp.min(jnp.where(lg == m1, eio, E), axis=1, keepdims=True)
    lg2 = jnp.where(eio == i1, -jnp.inf, lg)
    m2 = jnp.max(lg2, axis=1, keepdims=True)
    i2 = jnp.min(jnp.where(lg2 == m2, eio, E), axis=1, keepdims=True)
    p = jnp.exp(lg - m1)
    p = p / jnp.sum(p, axis=1, keepdims=True)
    sel1 = eio == i1
    sel2 = eio == i2
    g1 = jnp.sum(jnp.where(sel1, p, 0.0), axis=1, keepdims=True)
    g2 = jnp.sum(jnp.where(sel2, p, 0.0), axis=1, keepdims=True)
    sel_ref[...] = (sel1 | sel2).astype(jnp.float32)
    ids_ref[...] = jnp.concatenate([i1, i2], axis=1)
    gw_ref[...] = jnp.concatenate([g1, g2], axis=1)


_router = pl.pallas_call(
    _router_body,
    out_shape=(
        jax.ShapeDtypeStruct((T, E), jnp.float32),
        jax.ShapeDtypeStruct((T, K), jnp.int32),
        jax.ShapeDtypeStruct((T, K), jnp.float32),
    ),
)


TC = T // NW
DC = TC // 16


@functools.partial(
    pl.kernel,
    out_type=jax.ShapeDtypeStruct((P, H), jnp.float32),
    mesh=plsc.VectorSubcoreMesh(**_MESH),
    scratch_types=[
        pltpu.VMEM((DC, 16), jnp.int32),
        pltpu.VMEM((DC, 16), jnp.int32),
        pltpu.VMEM((2, 16, H), jnp.float32),
        pltpu.SemaphoreType.DMA,
        pltpu.SemaphoreType.DMA,
    ],
)
def _dispatch(hs_hbm, slot0_hbm, slot1_hbm, xs_hbm, slot0_v, slot1_v, buf,
              sem_g, sem_s):
    wid = lax.axis_index("s") * NC + lax.axis_index("c")
    pltpu.sync_copy(slot0_hbm.at[wid], slot0_v)
    pltpu.sync_copy(slot1_hbm.at[wid], slot1_v)
    g = [None] * DC
    s0 = [None] * DC
    s1 = [None] * DC
    base = wid * TC
    g[0] = pltpu.async_copy(hs_hbm.at[pl.ds(base, 16)], buf.at[0], sem_g)
    for j in range(DC):
        if j + 1 < DC:
            if j >= 1:
                s0[j - 1].wait()
                s1[j - 1].wait()
            g[j + 1] = pltpu.async_copy(
                hs_hbm.at[pl.ds(base + (j + 1) * 16, 16)],
                buf.at[(j + 1) % 2], sem_g)
        g[j].wait()
        s0[j] = pltpu.async_copy(buf.at[j % 2], xs_hbm.at[slot0_v.at[j]],
                                 sem_s)
        s1[j] = pltpu.async_copy(buf.at[j % 2], xs_hbm.at[slot1_v.at[j]],
                                 sem_s)
    s0[DC - 2].wait()
    s1[DC - 2].wait()
    s0[DC - 1].wait()
    s1[DC - 1].wait()


def _ffn_body(be_ref, xi_ref, na_ref, x_ref, w1_ref, w3_ref, w2_ref, y_ref):
    i = pl.program_id(0)

    @pl.when(i < na_ref[0])
    def _():
        x = x_ref[...]
        a = jnp.dot(x, w1_ref[0], preferred_element_type=jnp.float32)
        b = jnp.dot(x, w3_ref[0], preferred_element_type=jnp.float32)
        h = a * lax.logistic(a) * b
        y_ref[...] = jnp.dot(h, w2_ref[0], preferred_element_type=jnp.float32)


_ffn = pl.pallas_call(
    _ffn_body,
    grid_spec=pltpu.PrefetchScalarGridSpec(
        num_scalar_prefetch=3,
        grid=(NB,),
        in_specs=[
            pl.BlockSpec((BLK, H), lambda i, be, xi, na: (xi[i], 0)),
            pl.BlockSpec((1, H, I), lambda i, be, xi, na: (be[i], 0, 0)),
            pl.BlockSpec((1, H, I), lambda i, be, xi, na: (be[i], 0, 0)),
            pl.BlockSpec((1, I, H), lambda i, be, xi, na: (be[i], 0, 0)),
        ],
        out_specs=pl.BlockSpec((BLK, H), lambda i, be, xi, na: (i, 0)),
    ),
    out_shape=jax.ShapeDtypeStruct((P, H), jnp.float32),
)


@functools.partial(
    pl.kernel,
    out_type=jax.ShapeDtypeStruct((T, H), jnp.float32),
    mesh=plsc.VectorSubcoreMesh(**_MESH),
    scratch_types=[
        pltpu.VMEM((CH, 16), jnp.int32),
        pltpu.VMEM((CH, 16), jnp.float32),
        pltpu.VMEM((2, 16, H), jnp.float32),
        pltpu.VMEM((2, 8, H), jnp.float32),
        pltpu.SemaphoreType.DMA,
        pltpu.SemaphoreType.DMA,
    ],
)
def _combine(y_hbm, slot_hbm, gate_hbm, out_hbm, slot_v, g_v, ybuf, obuf,
             sem_y, sem_o):
    wid = lax.axis_index("s") * NC + lax.axis_index("c")
    pltpu.sync_copy(slot_hbm.at[wid], slot_v)
    pltpu.sync_copy(gate_hbm.at[wid], g_v)
    yd = [None] * CH
    od = [None] * CH
    yd[0] = pltpu.async_copy(y_hbm.at[slot_v.at[0]], ybuf.at[0], sem_y)
    for j in range(CH):
        if j + 1 < CH:
            yd[j + 1] = pltpu.async_copy(
                y_hbm.at[slot_v.at[j + 1]], ybuf.at[(j + 1) % 2], sem_y)
        yd[j].wait()
        if j >= 2:
            od[j - 2].wait()
        jb = j % 2
        gvec = g_v[j]
        gs = [gvec[i] for i in range(16)]

        def body(c, _, jb=jb, gs=gs):
            sl = pl.ds(c * 16, 16)
            for r in range(8):
                obuf[jb, r, sl] = (gs[2 * r] * ybuf[jb, 2 * r, sl]
                                   + gs[2 * r + 1] * ybuf[jb, 2 * r + 1, sl])
            return 0

        lax.fori_loop(0, H // 16, body, 0)
        od[j] = pltpu.async_copy(
            obuf.at[jb], out_hbm.at[pl.ds(wid * (T // NW) + j * 8, 8)], sem_o)
    od[CH - 2].wait()
    od[CH - 1].wait()


def kernel(hidden_states, router_logits, w1, w3, w2):
    sel, ids, gw = _router(router_logits)

    rank = jnp.take_along_axis(jnp.cumsum(sel, axis=0) - sel, ids, axis=1)
    counts = jnp.sum(sel, axis=0).astype(jnp.int32)
    nblk = (counts + BLK - 1) // BLK
    blk_end = jnp.cumsum(nblk)
    blk_start = blk_end - nblk
    num_active = blk_end[-1]
    j_eff = jnp.minimum(jnp.arange(NB, dtype=jnp.int32), num_active - 1)
    block_expert = jnp.searchsorted(blk_end, j_eff, side="right").astype(
        jnp.int32)
    slot = blk_start[ids] * BLK + rank.astype(jnp.int32)

    slot3 = slot.reshape(NW, CH, 16)
    slot0_3 = slot[:, 0].reshape(NW, DC, 16)
    slot1_3 = slot[:, 1].reshape(NW, DC, 16)
    gate3 = gw.reshape(NW, CH, 16)

    x_sorted = _dispatch(hidden_states, slot0_3, slot1_3)
    y = _ffn(block_expert, j_eff, num_active[None], x_sorted, w1, w3, w2)
    return _combine(y, slot3, gate3)


# speedup vs baseline: 1.3984x; 1.0770x over previous
import functools

import jax
import jax.numpy as jnp
from jax import lax
from jax.experimental import pallas as pl
from jax.experimental.pallas import tpu as pltpu
from jax.experimental.pallas import tpu_sc as plsc

E = 8
K = 2
T = 2048
H = 2048
I = 1024
BLK = 256
NB = (T * K) // BLK + E
P = NB * BLK
NC = 2
NS = 16
NW = NC * NS
A = T * K
CH = A // (NW * 16)

_MESH = dict(core_axis_name="c", subcore_axis_name="s", num_cores=NC,
             num_subcores=NS)


def _router_body(logits_ref, slot_ref, gw_ref, be_ref, xi_ref, na_ref):
    lg = logits_ref[...]
    eio = lax.broadcasted_iota(jnp.int32, (T, E), 1)
    m1 = jnp.max(lg, axis=1, keepdims=True)
    i1 = jnp.min(jnp.where(lg == m1, eio, E), axis=1, keepdims=True)
    lg2 = jnp.where(eio == i1, -jnp.inf, lg)
    m2 = jnp.max(lg2, axis=1, keepdims=True)
    i2 = jnp.min(jnp.where(lg2 == m2, eio, E), axis=1, keepdims=True)
    p = jnp.exp(lg - m1)
    p = p / jnp.sum(p, axis=1, keepdims=True)
    sel1 = eio == i1
    sel2 = eio == i2
    g1 = jnp.sum(jnp.where(sel1, p, 0.0), axis=1, keepdims=True)
    g2 = jnp.sum(jnp.where(sel2, p, 0.0), axis=1, keepdims=True)
    sel = (sel1 | sel2).astype(jnp.float32)

    tio_r = lax.broadcasted_iota(jnp.int32, (T, T), 0)
    tio_c = lax.broadcasted_iota(jnp.int32, (T, T), 1)
    tri = (tio_c < tio_r).astype(jnp.float32)
    rank = jnp.dot(tri, sel, preferred_element_type=jnp.float32)

    counts = jnp.sum(sel, axis=0, keepdims=True)
    nblk = jnp.ceil(counts * (1.0 / BLK))
    triu8_r = lax.broadcasted_iota(jnp.int32, (E, E), 0)
    triu8_c = lax.broadcasted_iota(jnp.int32, (E, E), 1)
    triu8 = (triu8_r <= triu8_c).astype(jnp.float32)
    blk_end = jnp.dot(nblk, triu8, preferred_element_type=jnp.float32)
    blk_start = blk_end - nblk

    na = blk_end[:, E - 1:E].astype(jnp.int32)
    jcol = lax.broadcasted_iota(jnp.int32, (NB, 1), 0)
    j_eff = jnp.minimum(jcol, na - 1)
    ge = (jnp.broadcast_to(blk_end, (NB, E))
          <= j_eff.astype(jnp.float32))
    be = jnp.sum(ge.astype(jnp.int32), axis=1, keepdims=True)

    pad_off = blk_start * BLK + rank
    s1 = jnp.sum(jnp.where(sel1, pad_off, 0.0), axis=1, keepdims=True)
    s2 = jnp.sum(jnp.where(sel2, pad_off, 0.0), axis=1, keepdims=True)
    slot_ref[...] = jnp.concatenate([s1, s2], axis=1).astype(jnp.int32)
    gw_ref[...] = jnp.concatenate([g1, g2], axis=1)
    be_ref[...] = be
    xi_ref[...] = j_eff
    na_ref[...] = na


_router = pl.pallas_call(
    _router_body,
    out_shape=(
        jax.ShapeDtypeStruct((T, K), jnp.int32),
        jax.ShapeDtypeStruct((T, K), jnp.float32),
        jax.ShapeDtypeStruct((NB, 1), jnp.int32),
        jax.ShapeDtypeStruct((NB, 1), jnp.int32),
        jax.ShapeDtypeStruct((1, 1), jnp.int32),
    ),
)


TC = T // NW
DC = TC // 16


@functools.partial(
    pl.kernel,
    out_type=jax.ShapeDtypeStruct((P, H), jnp.float32),
    mesh=plsc.VectorSubcoreMesh(**_MESH),
    scratch_types=[
        pltpu.VMEM((DC, 16), jnp.int32),
        pltpu.VMEM((DC, 16), jnp.int32),
        pltpu.VMEM((2, 16, H), jnp.float32),
        pltpu.SemaphoreType.DMA,
        pltpu.SemaphoreType.DMA,
    ],
)
def _dispatch(hs_hbm, slot0_hbm, slot1_hbm, xs_hbm, slot0_v, slot1_v, buf,
              sem_g, sem_s):
    wid = lax.axis_index("s") * NC + lax.axis_index("c")
    pltpu.sync_copy(slot0_hbm.at[wid], slot0_v)
    pltpu.sync_copy(slot1_hbm.at[wid], slot1_v)
    g = [None] * DC
    s0 = [None] * DC
    s1 = [None] * DC
    base = wid * TC
    g[0] = pltpu.async_copy(hs_hbm.at[pl.ds(base, 16)], buf.at[0], sem_g)
    for j in range(DC):
        if j + 1 < DC:
            if j >= 1:
                s0[j - 1].wait()
                s1[j - 1].wait()
            g[j + 1] = pltpu.async_copy(
                hs_hbm.at[pl.ds(base + (j + 1) * 16, 16)],
                buf.at[(j + 1) % 2], sem_g)
        g[j].wait()
        s0[j] = pltpu.async_copy(buf.at[j % 2], xs_hbm.at[slot0_v.at[j]],
                                 sem_s)
        s1[j] = pltpu.async_copy(buf.at[j % 2], xs_hbm.at[slot1_v.at[j]],
                                 sem_s)
    s0[DC - 2].wait()
    s1[DC - 2].wait()
    s0[DC - 1].wait()
    s1[DC - 1].wait()


def _ffn_body(be_ref, xi_ref, na_ref, x_ref, w1_ref, w3_ref, w2_ref, y_ref):
    i = pl.program_id(0)

    @pl.when(i < na_ref[0])
    def _():
        x = x_ref[...]
        a = jnp.dot(x, w1_ref[0], preferred_element_type=jnp.float32)
        b = jnp.dot(x, w3_ref[0], preferred_element_type=jnp.float32)
        h = a * lax.logistic(a) * b
        y_ref[...] = jnp.dot(h, w2_ref[0], preferred_element_type=jnp.float32)


_ffn = pl.pallas_call(
    _ffn_body,
    grid_spec=pltpu.PrefetchScalarGridSpec(
        num_scalar_prefetch=3,
        grid=(NB,),
        in_specs=[
            pl.BlockSpec((BLK, H), lambda i, be, xi, na: (xi[i], 0)),
            pl.BlockSpec((1, H, I), lambda i, be, xi, na: (be[i], 0, 0)),
            pl.BlockSpec((1, H, I), lambda i, be, xi, na: (be[i], 0, 0)),
            pl.BlockSpec((1, I, H), lambda i, be, xi, na: (be[i], 0, 0)),
        ],
        out_specs=pl.BlockSpec((BLK, H), lambda i, be, xi, na: (i, 0)),
    ),
    out_shape=jax.ShapeDtypeStruct((P, H), jnp.float32),
)


@functools.partial(
    pl.kernel,
    out_type=jax.ShapeDtypeStruct((T, H), jnp.float32),
    mesh=plsc.VectorSubcoreMesh(**_MESH),
    scratch_types=[
        pltpu.VMEM((CH, 16), jnp.int32),
        pltpu.VMEM((CH, 16), jnp.float32),
        pltpu.VMEM((2, 16, H), jnp.float32),
        pltpu.VMEM((2, 8, H), jnp.float32),
        pltpu.SemaphoreType.DMA,
        pltpu.SemaphoreType.DMA,
    ],
)
def _combine(y_hbm, slot_hbm, gate_hbm, out_hbm, slot_v, g_v, ybuf, obuf,
             sem_y, sem_o):
    wid = lax.axis_index("s") * NC + lax.axis_index("c")
    pltpu.sync_copy(slot_hbm.at[wid], slot_v)
    pltpu.sync_copy(gate_hbm.at[wid], g_v)
    yd = [None] * CH
    od = [None] * CH
    yd[0] = pltpu.async_copy(y_hbm.at[slot_v.at[0]], ybuf.at[0], sem_y)
    for j in range(CH):
        if j + 1 < CH:
            yd[j + 1] = pltpu.async_copy(
                y_hbm.at[slot_v.at[j + 1]], ybuf.at[(j + 1) % 2], sem_y)
        yd[j].wait()
        if j >= 2:
            od[j - 2].wait()
        jb = j % 2
        gvec = g_v[j]
        gs = [gvec[i] for i in range(16)]

        def body(c, _, jb=jb, gs=gs):
            sl = pl.ds(c * 16, 16)
            for r in range(8):
                obuf[jb, r, sl] = (gs[2 * r] * ybuf[jb, 2 * r, sl]
                                   + gs[2 * r + 1] * ybuf[jb, 2 * r + 1, sl])
            return 0

        lax.fori_loop(0, H // 16, body, 0)
        od[j] = pltpu.async_copy(
            obuf.at[jb], out_hbm.at[pl.ds(wid * (T // NW) + j * 8, 8)], sem_o)
    od[CH - 2].wait()
    od[CH - 1].wait()


def kernel(hidden_states, router_logits, w1, w3, w2):
    slot, gw, be, xi, na = _router(router_logits)

    slot3 = slot.reshape(NW, CH, 16)
    slot0_3 = slot[:, 0].reshape(NW, DC, 16)
    slot1_3 = slot[:, 1].reshape(NW, DC, 16)
    gate3 = gw.reshape(NW, CH, 16)

    x_sorted = _dispatch(hidden_states, slot0_3, slot1_3)
    y = _ffn(be.reshape(NB), xi.reshape(NB), na.reshape(1), x_sorted,
             w1, w3, w2)
    return _combine(y, slot3, gate3)


# speedup vs baseline: 1.4127x; 1.0102x over previous
import functools

import jax
import jax.numpy as jnp
from jax import lax
from jax.experimental import pallas as pl
from jax.experimental.pallas import tpu as pltpu
from jax.experimental.pallas import tpu_sc as plsc

E = 8
K = 2
T = 2048
H = 2048
I = 1024
BLK = 256
NB = (T * K) // BLK + E
P = NB * BLK
NC = 2
NS = 16
NW = NC * NS
A = T * K
CH = A // (NW * 16)

_MESH = dict(core_axis_name="c", subcore_axis_name="s", num_cores=NC,
             num_subcores=NS)


def _router_body(logits_ref, slot_ref, gw_ref, be_ref, xi_ref, na_ref):
    lg = logits_ref[...]
    eio = lax.broadcasted_iota(jnp.int32, (T, E), 1)
    m1 = jnp.max(lg, axis=1, keepdims=True)
    i1 = jnp.min(jnp.where(lg == m1, eio, E), axis=1, keepdims=True)
    lg2 = jnp.where(eio == i1, -jnp.inf, lg)
    m2 = jnp.max(lg2, axis=1, keepdims=True)
    i2 = jnp.min(jnp.where(lg2 == m2, eio, E), axis=1, keepdims=True)
    p = jnp.exp(lg - m1)
    p = p / jnp.sum(p, axis=1, keepdims=True)
    sel1 = eio == i1
    sel2 = eio == i2
    g1 = jnp.sum(jnp.where(sel1, p, 0.0), axis=1, keepdims=True)
    g2 = jnp.sum(jnp.where(sel2, p, 0.0), axis=1, keepdims=True)
    sel = (sel1 | sel2).astype(jnp.float32)

    tio_r = lax.broadcasted_iota(jnp.int32, (T, T), 0)
    tio_c = lax.broadcasted_iota(jnp.int32, (T, T), 1)
    tri = (tio_c < tio_r).astype(jnp.float32)
    rank = jnp.dot(tri, sel, preferred_element_type=jnp.float32)

    counts = jnp.sum(sel, axis=0, keepdims=True)
    nblk = jnp.ceil(counts * (1.0 / BLK))
    triu8_r = lax.broadcasted_iota(jnp.int32, (E, E), 0)
    triu8_c = lax.broadcasted_iota(jnp.int32, (E, E), 1)
    triu8 = (triu8_r <= triu8_c).astype(jnp.float32)
    blk_end = jnp.dot(nblk, triu8, preferred_element_type=jnp.float32)
    blk_start = blk_end - nblk

    na = blk_end[:, E - 1:E].astype(jnp.int32)
    jcol = lax.broadcasted_iota(jnp.int32, (NB, 1), 0)
    j_eff = jnp.minimum(jcol, na - 1)
    ge = (jnp.broadcast_to(blk_end, (NB, E))
          <= j_eff.astype(jnp.float32))
    be = jnp.sum(ge.astype(jnp.int32), axis=1, keepdims=True)

    pad_off = blk_start * BLK + rank
    s1 = jnp.sum(jnp.where(sel1, pad_off, 0.0), axis=1, keepdims=True)
    s2 = jnp.sum(jnp.where(sel2, pad_off, 0.0), axis=1, keepdims=True)
    slot_ref[...] = jnp.concatenate([s1, s2], axis=1).astype(jnp.int32)
    gw_ref[...] = jnp.concatenate([g1, g2], axis=1)
    be_ref[...] = be
    xi_ref[...] = j_eff
    na_ref[...] = na


_router = pl.pallas_call(
    _router_body,
    out_shape=(
        jax.ShapeDtypeStruct((T, K), jnp.int32),
        jax.ShapeDtypeStruct((T, K), jnp.float32),
        jax.ShapeDtypeStruct((NB, 1), jnp.int32),
        jax.ShapeDtypeStruct((NB, 1), jnp.int32),
        jax.ShapeDtypeStruct((1, 1), jnp.int32),
    ),
)


TC = T // NW
DC = TC // 16


@functools.partial(
    pl.kernel,
    out_type=jax.ShapeDtypeStruct((P, H), jnp.float32),
    mesh=plsc.VectorSubcoreMesh(**_MESH),
    scratch_types=[
        pltpu.VMEM((DC, 16), jnp.int32),
        pltpu.VMEM((DC, 16), jnp.int32),
        pltpu.VMEM((2, 16, H), jnp.float32),
        pltpu.SemaphoreType.DMA,
        pltpu.SemaphoreType.DMA,
    ],
)
def _dispatch(hs_hbm, slot0_hbm, slot1_hbm, xs_hbm, slot0_v, slot1_v, buf,
              sem_g, sem_s):
    wid = lax.axis_index("s") * NC + lax.axis_index("c")
    pltpu.sync_copy(slot0_hbm.at[wid], slot0_v)
    pltpu.sync_copy(slot1_hbm.at[wid], slot1_v)
    g = [None] * DC
    s0 = [None] * DC
    s1 = [None] * DC
    base = wid * TC
    g[0] = pltpu.async_copy(hs_hbm.at[pl.ds(base, 16)], buf.at[0], sem_g)
    for j in range(DC):
        if j + 1 < DC:
            if j >= 1:
                s0[j - 1].wait()
                s1[j - 1].wait()
            g[j + 1] = pltpu.async_copy(
                hs_hbm.at[pl.ds(base + (j + 1) * 16, 16)],
                buf.at[(j + 1) % 2], sem_g)
        g[j].wait()
        s0[j] = pltpu.async_copy(buf.at[j % 2], xs_hbm.at[slot0_v.at[j]],
                                 sem_s)
        s1[j] = pltpu.async_copy(buf.at[j % 2], xs_hbm.at[slot1_v.at[j]],
                                 sem_s)
    s0[DC - 2].wait()
    s1[DC - 2].wait()
    s0[DC - 1].wait()
    s1[DC - 1].wait()


def _ffn_body(be_ref, xi_ref, na_ref, x_ref, w1_ref, w3_ref, w2_ref, y_ref):
    i = pl.program_id(0)

    @pl.when(i < na_ref[0])
    def _():
        x = x_ref[...]
        a = jnp.dot(x, w1_ref[0], preferred_element_type=jnp.float32)
        b = jnp.dot(x, w3_ref[0], preferred_element_type=jnp.float32)
        h = a * lax.logistic(a) * b
        y_ref[...] = jnp.dot(h, w2_ref[0], preferred_element_type=jnp.float32)


_ffn = pl.pallas_call(
    _ffn_body,
    grid_spec=pltpu.PrefetchScalarGridSpec(
        num_scalar_prefetch=3,
        grid=(NB,),
        in_specs=[
            pl.BlockSpec((BLK, H), lambda i, be, xi, na: (xi[i], 0)),
            pl.BlockSpec((1, H, I), lambda i, be, xi, na: (be[i], 0, 0)),
            pl.BlockSpec((1, H, I), lambda i, be, xi, na: (be[i], 0, 0)),
            pl.BlockSpec((1, I, H), lambda i, be, xi, na: (be[i], 0, 0)),
        ],
        out_specs=pl.BlockSpec((BLK, H), lambda i, be, xi, na: (xi[i], 0)),
    ),
    out_shape=jax.ShapeDtypeStruct((P, H), jnp.float32),
)


@functools.partial(
    pl.kernel,
    out_type=jax.ShapeDtypeStruct((T, H), jnp.float32),
    mesh=plsc.VectorSubcoreMesh(**_MESH),
    scratch_types=[
        pltpu.VMEM((CH, 16), jnp.int32),
        pltpu.VMEM((CH, 16), jnp.float32),
        pltpu.VMEM((2, 16, H), jnp.float32),
        pltpu.VMEM((2, 8, H), jnp.float32),
        pltpu.SemaphoreType.DMA,
        pltpu.SemaphoreType.DMA,
    ],
)
def _combine(y_hbm, slot_hbm, gate_hbm, out_hbm, slot_v, g_v, ybuf, obuf,
             sem_y, sem_o):
    wid = lax.axis_index("s") * NC + lax.axis_index("c")
    pltpu.sync_copy(slot_hbm.at[wid], slot_v)
    pltpu.sync_copy(gate_hbm.at[wid], g_v)
    yd = [None] * CH
    od = [None] * CH
    yd[0] = pltpu.async_copy(y_hbm.at[slot_v.at[0]], ybuf.at[0], sem_y)
    for j in range(CH):
        if j + 1 < CH:
            yd[j + 1] = pltpu.async_copy(
                y_hbm.at[slot_v.at[j + 1]], ybuf.at[(j + 1) % 2], sem_y)
        yd[j].wait()
        if j >= 2:
            od[j - 2].wait()
        jb = j % 2
        gvec = g_v[j]
        gs = [gvec[i] for i in range(16)]

        def body(c, _, jb=jb, gs=gs):
            sl = pl.ds(c * 16, 16)
            for r in range(8):
                obuf[jb, r, sl] = (gs[2 * r] * ybuf[jb, 2 * r, sl]
                                   + gs[2 * r + 1] * ybuf[jb, 2 * r + 1, sl])
            return 0

        lax.fori_loop(0, H // 16, body, 0)
        od[j] = pltpu.async_copy(
            obuf.at[jb], out_hbm.at[pl.ds(wid * (T // NW) + j * 8, 8)], sem_o)
    od[CH - 2].wait()
    od[CH - 1].wait()


def kernel(hidden_states, router_logits, w1, w3, w2):
    slot, gw, be, xi, na = _router(router_logits)

    slot3 = slot.reshape(NW, CH, 16)
    slot0_3 = slot[:, 0].reshape(NW, DC, 16)
    slot1_3 = slot[:, 1].reshape(NW, DC, 16)
    gate3 = gw.reshape(NW, CH, 16)

    x_sorted = _dispatch(hidden_states, slot0_3, slot1_3)
    y = _ffn(be.reshape(NB), xi.reshape(NB), na.reshape(1), x_sorted,
             w1, w3, w2)
    return _combine(y, slot3, gate3)
